# baseline (speedup 1.0000x reference)
import functools

import jax
import jax.numpy as jnp
from jax import lax
from jax.experimental import pallas as pl
from jax.experimental.pallas import tpu as pltpu
from jax.experimental.pallas import tpu_sc as plsc

N_ROWS = 320000
D_FEAT = 128
NUM_SEGMENTS = 10000

NC = 2
NS = 16
NW = NC * NS
SEG_RANGE = 320
LAST_SEGS = NUM_SEGMENTS - (NW - 1) * SEG_RANGE
ACC_ROWS = SEG_RANGE + 8
TRASH_ROW = SEG_RANGE + 1
CHUNK = 160
THR_STEP = 80


_PRE_BLOCK = 104
_PRE_GRID = 25
_PRE_ROWS = _PRE_BLOCK * _PRE_GRID


def _bounds_body(idx_ref, out_ref):
  blk = idx_ref[...]
  thr = lax.broadcasted_iota(jnp.int32, (1, 1, D_FEAT), 2) * THR_STEP
  m = (blk[:, :, None] < thr).astype(jnp.int32)
  s = jnp.sum(m, axis=(0, 1))[None, :]

  @pl.when(pl.program_id(0) == 0)
  def _():
    out_ref[...] = jnp.zeros_like(out_ref)

  out_ref[...] += s


def _row_bounds(study_indexes):
  pad = _PRE_ROWS * D_FEAT - N_ROWS
  idx2d = jnp.concatenate(
      [study_indexes, jnp.full((pad,), jnp.int32(2**30), jnp.int32)]
  ).reshape(_PRE_ROWS, D_FEAT)
  counts = pl.pallas_call(
      _bounds_body,
      grid=(_PRE_GRID,),
      in_specs=[pl.BlockSpec((_PRE_BLOCK, D_FEAT), lambda i: (i, 0))],
      out_specs=pl.BlockSpec((1, D_FEAT), lambda i: (0, 0)),
      out_shape=jax.ShapeDtypeStruct((1, D_FEAT), jnp.int32),
  )(idx2d)
  b = counts[0, 0 :: SEG_RANGE // THR_STEP]
  b = jnp.concatenate([b, jnp.full((2,), N_ROWS, jnp.int32)])
  return jnp.repeat(b, 16)


def _sc_segment_mean(embeddings, study_indexes, bounds):
  mesh = plsc.VectorSubcoreMesh(
      core_axis_name="c", subcore_axis_name="s", num_cores=NC, num_subcores=NS
  )

  @functools.partial(
      pl.kernel,
      out_type=jax.ShapeDtypeStruct((NUM_SEGMENTS * D_FEAT,), jnp.float32),
      mesh=mesh,
      compiler_params=pltpu.CompilerParams(needs_layout_passes=False),
      scratch_types=[
          pltpu.VMEM((544,), jnp.int32),
          pltpu.VMEM((CHUNK,), jnp.int32),
          pltpu.VMEM((CHUNK,), jnp.int32),
          pltpu.VMEM((CHUNK * D_FEAT,), jnp.float32),
          pltpu.VMEM((CHUNK * D_FEAT,), jnp.float32),
          pltpu.VMEM((ACC_ROWS * D_FEAT,), jnp.float32),
          pltpu.VMEM((336,), jnp.float32),
          pltpu.SemaphoreType.DMA,
          pltpu.SemaphoreType.DMA,
      ],
  )
  def k(emb_hbm, idx_hbm, bnd_hbm, out_hbm,
        bvm, idx_buf0, idx_buf1, row_buf0, row_buf1, acc, cnt, sem0, sem1):
    idx_bufs = (idx_buf0, idx_buf1)
    row_bufs = (row_buf0, row_buf1)
    sems = (sem0, sem1)
    c = lax.axis_index("c")
    s = lax.axis_index("s")
    t = c * NS + s
    seg_base = t * SEG_RANGE

    pltpu.sync_copy(bnd_hbm, bvm)

    zeros16 = jnp.zeros((16,), jnp.float32)
    ones16 = jnp.full((16,), 1.0, jnp.float32)
    iota16 = lax.broadcasted_iota(jnp.int32, (16,), 0)

    def init_acc(i, _):
      acc[pl.ds(i * 16, 16)] = zeros16
      return 0

    lax.fori_loop(0, ACC_ROWS * D_FEAT // 16, init_acc, 0)

    def init_cnt(i, _):
      cnt[pl.ds(i * 16, 16)] = zeros16
      return 0

    lax.fori_loop(0, 336 // 16, init_cnt, 0)

    def read_bound(j):
      blk = bvm[pl.ds(pl.multiple_of(j * 16, 16), 16)]
      return blk[0]

    b_lo = read_bound(t)
    b_hi = read_bound(t + 1)
    lo8 = pl.multiple_of(b_lo - lax.rem(b_lo, 8), 8)
    span = b_hi - lo8
    nch = lax.div(span + CHUNK - 1, CHUNK)

    def chunk_base(cid):
      start = lo8 + cid * CHUNK
      base = pl.multiple_of(jnp.minimum(start, N_ROWS - CHUNK), 8)
      return start, base

    def issue(cid, b):
      _, base = chunk_base(cid)
      pltpu.async_copy(idx_hbm.at[pl.ds(base, CHUNK)], idx_bufs[b], sems[b])
      pltpu.async_copy(
          emb_hbm.at[pl.ds(base * D_FEAT, CHUNK * D_FEAT)], row_bufs[b], sems[b]
      )

    def wait(cid, b):
      _, base = chunk_base(cid)
      pltpu.make_async_copy(idx_hbm.at[pl.ds(base, CHUNK)], idx_bufs[b],
                            sems[b]).wait()
      pltpu.make_async_copy(
          emb_hbm.at[pl.ds(base * D_FEAT, CHUNK * D_FEAT)], row_bufs[b], sems[b]
      ).wait()

    def compute(cid, b):
      start, base = chunk_base(cid)
      idx_buf = idx_bufs[b]
      row_buf = row_bufs[b]

      def halfloop(h, _):
        gb = h * (CHUNK // 32)
        lvs = []
        for k in range(CHUNK // 32):
          g = gb + k
          v = idx_buf[pl.ds(pl.multiple_of(g * 16, 16), 16)]
          local = v - seg_base
          pos = base + g * 16 + iota16
          bad = (local < 0) | (local >= SEG_RANGE) | (pos < start)
          lv = jnp.where(bad, TRASH_ROW, local)
          plsc.addupdate_scatter(cnt, [lv], ones16)
          lvs.append((lv, g))
        for j in range(16):
          for lv, g in lvs:
            off = lv[j] * D_FEAT
            rbase = (g * 16 + j) * D_FEAT
            for q in range(D_FEAT // 16):
              vals = row_buf[pl.ds(rbase + q * 16, 16)]
              plsc.addupdate(acc.at[pl.ds(off + q * 16, 16)], vals)
        return 0

      lax.fori_loop(0, 2, halfloop, 0)

    @pl.when(nch > 0)
    def _():
      issue(0, 0)

    @pl.when(nch > 1)
    def _():
      issue(1, 1)

    def body(i2, _):
      for b in range(2):
        cid = i2 * 2 + b

        @pl.when(cid < nch)
        def _():
          wait(cid, b)
          compute(cid, b)

          @pl.when(cid + 2 < nch)
          def _():
            issue(cid + 2, b)

      return 0

    lax.fori_loop(0, lax.div(nch + 1, 2), body, 0)

    def mean_grp(g, _):
      cv = cnt[pl.ds(g * 16, 16)]
      rec16 = 1.0 / jnp.maximum(cv, 1.0)
      for j in range(16):
        rbase = (g * 16 + j) * D_FEAT
        rec = rec16[j]
        for q in range(D_FEAT // 16):
          acc[pl.ds(rbase + q * 16, 16)] = acc[pl.ds(rbase + q * 16, 16)] * rec
      return 0

    lax.fori_loop(0, SEG_RANGE // 16, mean_grp, 0)

    @pl.when(t < NW - 1)
    def _():
      pltpu.sync_copy(acc.at[pl.ds(0, SEG_RANGE * D_FEAT)],
                      out_hbm.at[pl.ds(seg_base * D_FEAT, SEG_RANGE * D_FEAT)])

    @pl.when(t == NW - 1)
    def _():
      pltpu.sync_copy(acc.at[pl.ds(0, LAST_SEGS * D_FEAT)],
                      out_hbm.at[pl.ds(seg_base * D_FEAT, LAST_SEGS * D_FEAT)])

  return k(embeddings.reshape(-1), study_indexes, bounds)


@jax.jit
def kernel(embeddings, study_indexes):
  bounds = _row_bounds(study_indexes)
  out = _sc_segment_mean(embeddings, study_indexes, bounds)
  return out.reshape(NUM_SEGMENTS, D_FEAT)


# speedup vs baseline: 2.6453x; 2.6453x over previous
import functools

import jax
import jax.numpy as jnp
from jax import lax
from jax.experimental import pallas as pl
from jax.experimental.pallas import tpu as pltpu
from jax.experimental.pallas import tpu_sc as plsc

N_ROWS = 320000
D_FEAT = 128
NUM_SEGMENTS = 10000

NC = 2
NS = 16
NW = NC * NS
SEG_RANGE = 320
LAST_SEGS = NUM_SEGMENTS - (NW - 1) * SEG_RANGE
ACC_ROWS = SEG_RANGE + 8
TRASH_ROW = SEG_RANGE + 1
CHUNK = 160
THR_STEP = 80


_PRE_BLOCK = 104
_PRE_GRID = 25
_PRE_ROWS = _PRE_BLOCK * _PRE_GRID


def _bounds_body(idx_ref, out_ref):
  blk = idx_ref[...]
  thr = lax.broadcasted_iota(jnp.int32, (1, 1, D_FEAT), 2) * THR_STEP
  m = (blk[:, :, None] < thr).astype(jnp.int32)
  s = jnp.sum(m, axis=(0, 1))[None, :]

  @pl.when(pl.program_id(0) == 0)
  def _():
    out_ref[...] = jnp.zeros_like(out_ref)

  out_ref[...] += s


def _row_bounds(study_indexes):
  pad = _PRE_ROWS * D_FEAT - N_ROWS
  idx2d = jnp.concatenate(
      [study_indexes, jnp.full((pad,), jnp.int32(2**30), jnp.int32)]
  ).reshape(_PRE_ROWS, D_FEAT)
  counts = pl.pallas_call(
      _bounds_body,
      grid=(_PRE_GRID,),
      in_specs=[pl.BlockSpec((_PRE_BLOCK, D_FEAT), lambda i: (i, 0))],
      out_specs=pl.BlockSpec((1, D_FEAT), lambda i: (0, 0)),
      out_shape=jax.ShapeDtypeStruct((1, D_FEAT), jnp.int32),
  )(idx2d)
  b = counts[0, 0 :: SEG_RANGE // THR_STEP]
  b = jnp.concatenate([b, jnp.full((2,), N_ROWS, jnp.int32)])
  return jnp.repeat(b, 16)


def _sc_segment_mean(embeddings, study_indexes, bounds):
  mesh = plsc.VectorSubcoreMesh(
      core_axis_name="c", subcore_axis_name="s", num_cores=NC, num_subcores=NS
  )

  @functools.partial(
      pl.kernel,
      out_type=jax.ShapeDtypeStruct((NUM_SEGMENTS * D_FEAT,), jnp.float32),
      mesh=mesh,
      compiler_params=pltpu.CompilerParams(needs_layout_passes=False),
      scratch_types=[
          pltpu.VMEM((544,), jnp.int32),
          pltpu.VMEM((CHUNK,), jnp.int32),
          pltpu.VMEM((CHUNK,), jnp.int32),
          pltpu.VMEM((CHUNK * D_FEAT,), jnp.float32),
          pltpu.VMEM((CHUNK * D_FEAT,), jnp.float32),
          pltpu.VMEM((ACC_ROWS * D_FEAT,), jnp.float32),
          pltpu.VMEM((336,), jnp.float32),
          pltpu.SemaphoreType.DMA,
          pltpu.SemaphoreType.DMA,
      ],
  )
  def k(emb_hbm, idx_hbm, bnd_hbm, out_hbm,
        bvm, idx_buf0, idx_buf1, row_buf0, row_buf1, acc, cnt, sem0, sem1):
    idx_bufs = (idx_buf0, idx_buf1)
    row_bufs = (row_buf0, row_buf1)
    sems = (sem0, sem1)
    c = lax.axis_index("c")
    s = lax.axis_index("s")
    t = c * NS + s
    seg_base = t * SEG_RANGE

    pltpu.sync_copy(bnd_hbm, bvm)

    zeros16 = jnp.zeros((16,), jnp.float32)
    ones16 = jnp.full((16,), 1.0, jnp.float32)
    iota16 = lax.broadcasted_iota(jnp.int32, (16,), 0)

    def init_acc(i, _):
      acc[pl.ds(i * 16, 16)] = zeros16
      return 0

    lax.fori_loop(0, ACC_ROWS * D_FEAT // 16, init_acc, 0)

    def init_cnt(i, _):
      cnt[pl.ds(i * 16, 16)] = zeros16
      return 0

    lax.fori_loop(0, 336 // 16, init_cnt, 0)

    def read_bound(j):
      blk = bvm[pl.ds(pl.multiple_of(j * 16, 16), 16)]
      return blk[0]

    b_lo = read_bound(t)
    b_hi = read_bound(t + 1)
    lo8 = pl.multiple_of(b_lo - lax.rem(b_lo, 8), 8)
    span = b_hi - lo8
    nch = lax.div(span + CHUNK - 1, CHUNK)

    def chunk_base(cid):
      start = lo8 + cid * CHUNK
      base = pl.multiple_of(jnp.minimum(start, N_ROWS - CHUNK), 8)
      return start, base

    def issue(cid, b):
      _, base = chunk_base(cid)
      pltpu.async_copy(idx_hbm.at[pl.ds(base, CHUNK)], idx_bufs[b], sems[b])
      pltpu.async_copy(
          emb_hbm.at[pl.ds(base * D_FEAT, CHUNK * D_FEAT)], row_bufs[b], sems[b]
      )

    def wait(cid, b):
      _, base = chunk_base(cid)
      pltpu.make_async_copy(idx_hbm.at[pl.ds(base, CHUNK)], idx_bufs[b],
                            sems[b]).wait()
      pltpu.make_async_copy(
          emb_hbm.at[pl.ds(base * D_FEAT, CHUNK * D_FEAT)], row_bufs[b], sems[b]
      ).wait()

    def compute(cid, b):
      start, base = chunk_base(cid)
      idx_buf = idx_bufs[b]
      row_buf = row_bufs[b]

      def grploop(g, _):
        v = idx_buf[pl.ds(pl.multiple_of(g * 16, 16), 16)]
        local = v - seg_base
        pos = base + g * 16 + iota16
        bad = (local < 0) | (local >= SEG_RANGE) | (pos < start)
        lv = jnp.where(bad, TRASH_ROW, local)
        plsc.addupdate_scatter(cnt, [lv], ones16)
        gbase = g * (16 * D_FEAT)
        nq = D_FEAT // 16
        for j in range(0, 16, 2):
          offs = [lv[j] * D_FEAT, lv[j + 1] * D_FEAT]
          vals = []
          for dj in range(2):
            rbase = gbase + (j + dj) * D_FEAT
            vals.append(
                [row_buf[pl.ds(rbase + q * 16, 16)] for q in range(nq)])
          for dj in range(2):
            for q in range(nq):
              plsc.addupdate(acc.at[pl.ds(offs[dj] + q * 16, 16)],
                             vals[dj][q])
        return 0

      lax.fori_loop(0, CHUNK // 16, grploop, 0)

    @pl.when(nch > 0)
    def _():
      issue(0, 0)

    @pl.when(nch > 1)
    def _():
      issue(1, 1)

    def body(i2, _):
      for b in range(2):
        cid = i2 * 2 + b

        @pl.when(cid < nch)
        def _():
          wait(cid, b)
          compute(cid, b)

          @pl.when(cid + 2 < nch)
          def _():
            issue(cid + 2, b)

      return 0

    lax.fori_loop(0, lax.div(nch + 1, 2), body, 0)

    def mean_grp(g, _):
      cv = cnt[pl.ds(g * 16, 16)]
      rec16 = 1.0 / jnp.maximum(cv, 1.0)
      for j in range(16):
        rbase = (g * 16 + j) * D_FEAT
        rec = rec16[j]
        for q in range(D_FEAT // 16):
          acc[pl.ds(rbase + q * 16, 16)] = acc[pl.ds(rbase + q * 16, 16)] * rec
      return 0

    lax.fori_loop(0, SEG_RANGE // 16, mean_grp, 0)

    @pl.when(t < NW - 1)
    def _():
      pltpu.sync_copy(acc.at[pl.ds(0, SEG_RANGE * D_FEAT)],
                      out_hbm.at[pl.ds(seg_base * D_FEAT, SEG_RANGE * D_FEAT)])

    @pl.when(t == NW - 1)
    def _():
      pltpu.sync_copy(acc.at[pl.ds(0, LAST_SEGS * D_FEAT)],
                      out_hbm.at[pl.ds(seg_base * D_FEAT, LAST_SEGS * D_FEAT)])

  return k(embeddings.reshape(-1), study_indexes, bounds)


@jax.jit
def kernel(embeddings, study_indexes):
  bounds = _row_bounds(study_indexes)
  out = _sc_segment_mean(embeddings, study_indexes, bounds)
  return out.reshape(NUM_SEGMENTS, D_FEAT)


# speedup vs baseline: 2.6475x; 1.0008x over previous
import functools

import jax
import jax.numpy as jnp
from jax import lax
from jax.experimental import pallas as pl
from jax.experimental.pallas import tpu as pltpu
from jax.experimental.pallas import tpu_sc as plsc

N_ROWS = 320000
D_FEAT = 128
NUM_SEGMENTS = 10000

NC = 2
NS = 16
NW = NC * NS
SEG_RANGE = 320
LAST_SEGS = NUM_SEGMENTS - (NW - 1) * SEG_RANGE
ACC_ROWS = SEG_RANGE + 8
TRASH_ROW = SEG_RANGE + 1
CHUNK = 160
THR_STEP = 80


_PRE_BLOCK = 104
_PRE_GRID = 25
_PRE_ROWS = _PRE_BLOCK * _PRE_GRID


def _bounds_body(idx_ref, out_ref):
  blk = idx_ref[...]
  thr = lax.broadcasted_iota(jnp.int32, (1, 1, D_FEAT), 2) * THR_STEP
  m = (blk[:, :, None] < thr).astype(jnp.int32)
  s = jnp.sum(m, axis=(0, 1))[None, :]

  @pl.when(pl.program_id(0) == 0)
  def _():
    out_ref[...] = jnp.zeros_like(out_ref)

  out_ref[...] += s


def _row_bounds(study_indexes):
  pad = _PRE_ROWS * D_FEAT - N_ROWS
  idx2d = jnp.concatenate(
      [study_indexes, jnp.full((pad,), jnp.int32(2**30), jnp.int32)]
  ).reshape(_PRE_ROWS, D_FEAT)
  counts = pl.pallas_call(
      _bounds_body,
      grid=(_PRE_GRID,),
      in_specs=[pl.BlockSpec((_PRE_BLOCK, D_FEAT), lambda i: (i, 0))],
      out_specs=pl.BlockSpec((1, D_FEAT), lambda i: (0, 0)),
      out_shape=jax.ShapeDtypeStruct((1, D_FEAT), jnp.int32),
  )(idx2d)
  b = counts[0, 0 :: SEG_RANGE // THR_STEP]
  b = jnp.concatenate([b, jnp.full((2,), N_ROWS, jnp.int32)])
  return jnp.repeat(b, 16)


def _sc_segment_mean(embeddings, study_indexes, bounds):
  mesh = plsc.VectorSubcoreMesh(
      core_axis_name="c", subcore_axis_name="s", num_cores=NC, num_subcores=NS
  )

  @functools.partial(
      pl.kernel,
      out_type=jax.ShapeDtypeStruct((NUM_SEGMENTS * D_FEAT,), jnp.float32),
      mesh=mesh,
      compiler_params=pltpu.CompilerParams(needs_layout_passes=False),
      scratch_types=[
          pltpu.VMEM((544,), jnp.int32),
          pltpu.VMEM((CHUNK,), jnp.int32),
          pltpu.VMEM((CHUNK,), jnp.int32),
          pltpu.VMEM((CHUNK * D_FEAT,), jnp.float32),
          pltpu.VMEM((CHUNK * D_FEAT,), jnp.float32),
          pltpu.VMEM((ACC_ROWS * D_FEAT,), jnp.float32),
          pltpu.VMEM((336,), jnp.float32),
          pltpu.SemaphoreType.DMA,
          pltpu.SemaphoreType.DMA,
      ],
  )
  def k(emb_hbm, idx_hbm, bnd_hbm, out_hbm,
        bvm, idx_buf0, idx_buf1, row_buf0, row_buf1, acc, cnt, sem0, sem1):
    idx_bufs = (idx_buf0, idx_buf1)
    row_bufs = (row_buf0, row_buf1)
    sems = (sem0, sem1)
    c = lax.axis_index("c")
    s = lax.axis_index("s")
    t = c * NS + s
    seg_base = t * SEG_RANGE

    pltpu.sync_copy(bnd_hbm, bvm)

    zeros16 = jnp.zeros((16,), jnp.float32)
    ones16 = jnp.full((16,), 1.0, jnp.float32)
    iota16 = lax.broadcasted_iota(jnp.int32, (16,), 0)

    def init_acc(i, _):
      acc[pl.ds(i * 16, 16)] = zeros16
      return 0

    lax.fori_loop(0, ACC_ROWS * D_FEAT // 16, init_acc, 0)

    def init_cnt(i, _):
      cnt[pl.ds(i * 16, 16)] = zeros16
      return 0

    lax.fori_loop(0, 336 // 16, init_cnt, 0)

    def read_bound(j):
      blk = bvm[pl.ds(pl.multiple_of(j * 16, 16), 16)]
      return blk[0]

    b_lo = read_bound(t)
    b_hi = read_bound(t + 1)
    lo8 = pl.multiple_of(b_lo - lax.rem(b_lo, 8), 8)
    span = b_hi - lo8
    nch = lax.div(span + CHUNK - 1, CHUNK)

    def chunk_base(cid):
      start = lo8 + cid * CHUNK
      base = pl.multiple_of(jnp.minimum(start, N_ROWS - CHUNK), 8)
      return start, base

    def issue(cid, b):
      _, base = chunk_base(cid)
      pltpu.async_copy(idx_hbm.at[pl.ds(base, CHUNK)], idx_bufs[b], sems[b])
      pltpu.async_copy(
          emb_hbm.at[pl.ds(base * D_FEAT, CHUNK * D_FEAT)], row_bufs[b], sems[b]
      )

    def wait(cid, b):
      _, base = chunk_base(cid)
      pltpu.make_async_copy(idx_hbm.at[pl.ds(base, CHUNK)], idx_bufs[b],
                            sems[b]).wait()
      pltpu.make_async_copy(
          emb_hbm.at[pl.ds(base * D_FEAT, CHUNK * D_FEAT)], row_bufs[b], sems[b]
      ).wait()

    def compute(cid, b):
      start, base = chunk_base(cid)
      idx_buf = idx_bufs[b]
      row_buf = row_bufs[b]

      def grploop(g, _):
        v = idx_buf[pl.ds(pl.multiple_of(g * 16, 16), 16)]
        local = v - seg_base
        pos = base + g * 16 + iota16
        bad = (local < 0) | (local >= SEG_RANGE) | (pos < start)
        lv = jnp.where(bad, TRASH_ROW, local)
        plsc.addupdate_scatter(cnt, [lv], ones16)
        gbase = g * (16 * D_FEAT)
        nq = D_FEAT // 16
        for j in range(0, 16, 4):
          offs = [lv[j + dj] * D_FEAT for dj in range(4)]
          vals = []
          for dj in range(4):
            rbase = gbase + (j + dj) * D_FEAT
            vals.append(
                [row_buf[pl.ds(rbase + q * 16, 16)] for q in range(nq)])
          for dj in range(4):
            for q in range(nq):
              plsc.addupdate(acc.at[pl.ds(offs[dj] + q * 16, 16)],
                             vals[dj][q])
        return 0

      lax.fori_loop(0, CHUNK // 16, grploop, 0)

    @pl.when(nch > 0)
    def _():
      issue(0, 0)

    @pl.when(nch > 1)
    def _():
      issue(1, 1)

    def body(i2, _):
      for b in range(2):
        cid = i2 * 2 + b

        @pl.when(cid < nch)
        def _():
          wait(cid, b)
          compute(cid, b)

          @pl.when(cid + 2 < nch)
          def _():
            issue(cid + 2, b)

      return 0

    lax.fori_loop(0, lax.div(nch + 1, 2), body, 0)

    def mean_grp(g, _):
      cv = cnt[pl.ds(g * 16, 16)]
      rec16 = 1.0 / jnp.maximum(cv, 1.0)
      for j in range(16):
        rbase = (g * 16 + j) * D_FEAT
        rec = rec16[j]
        for q in range(D_FEAT // 16):
          acc[pl.ds(rbase + q * 16, 16)] = acc[pl.ds(rbase + q * 16, 16)] * rec
      return 0

    lax.fori_loop(0, SEG_RANGE // 16, mean_grp, 0)

    @pl.when(t < NW - 1)
    def _():
      pltpu.sync_copy(acc.at[pl.ds(0, SEG_RANGE * D_FEAT)],
                      out_hbm.at[pl.ds(seg_base * D_FEAT, SEG_RANGE * D_FEAT)])

    @pl.when(t == NW - 1)
    def _():
      pltpu.sync_copy(acc.at[pl.ds(0, LAST_SEGS * D_FEAT)],
                      out_hbm.at[pl.ds(seg_base * D_FEAT, LAST_SEGS * D_FEAT)])

  return k(embeddings.reshape(-1), study_indexes, bounds)


@jax.jit
def kernel(embeddings, study_indexes):
  bounds = _row_bounds(study_indexes)
  out = _sc_segment_mean(embeddings, study_indexes, bounds)
  return out.reshape(NUM_SEGMENTS, D_FEAT)
